# Initial kernel scaffold; baseline (speedup 1.0000x reference)
import jax
import jax.numpy as jnp
from jax.experimental import pallas as pl


def kernel(x, weight):
    raise NotImplementedError("write your pallas kernel here")


# speedup vs baseline: 1.5701x; 1.5701x over previous
import functools

import jax
import jax.numpy as jnp
from jax import lax
from jax.experimental import pallas as pl
from jax.experimental.pallas import tpu as pltpu
from jax.experimental.pallas import tpu_sc as plsc

_D = 32
_CHUNK = 128
_NBUF = 4
_NW = 32


def _sc_embedding_lookup(idx3, weight, n_chunks):
    b_per_w = n_chunks * _CHUNK
    b_total = _NW * b_per_w
    n_groups = n_chunks // _NBUF

    mesh = plsc.VectorSubcoreMesh(core_axis_name="c", subcore_axis_name="s")

    @functools.partial(
        pl.kernel,
        mesh=mesh,
        out_type=jax.ShapeDtypeStruct((b_total, _D), jnp.float32),
        scratch_types=[
            pltpu.VMEM((n_chunks, _CHUNK), jnp.int32),
            pltpu.VMEM((_NBUF, _CHUNK, _D), jnp.float32),
        ] + [pltpu.SemaphoreType.DMA] * _NBUF,
        compiler_params=pltpu.CompilerParams(use_tc_tiling_on_sc=False),
    )
    def k(table_hbm, idx_hbm, out_hbm, idx_v, rows_v, *sems):
        wid = lax.axis_index("s") * 2 + lax.axis_index("c")
        base = wid * b_per_w
        pltpu.sync_copy(idx_hbm.at[wid], idx_v)

        def out_slice(c):
            return out_hbm.at[pl.ds(base + c * _CHUNK, _CHUNK)]

        for b in range(_NBUF):
            pltpu.async_copy(table_hbm.at[idx_v.at[b]], rows_v.at[b], sems[b])

        def body(g, carry):
            for b in range(_NBUF):
                c = g * _NBUF + b
                pltpu.make_async_copy(
                    table_hbm.at[idx_v.at[c]], rows_v.at[b], sems[b]
                ).wait()
                pltpu.async_copy(rows_v.at[b], out_slice(c), sems[b])
                pltpu.make_async_copy(rows_v.at[b], out_slice(c), sems[b]).wait()
                pltpu.async_copy(
                    table_hbm.at[idx_v.at[c + _NBUF]], rows_v.at[b], sems[b]
                )
            return carry

        lax.fori_loop(0, n_groups - 1, body, 0)

        for b in range(_NBUF):
            c = (n_groups - 1) * _NBUF + b
            pltpu.make_async_copy(
                table_hbm.at[idx_v.at[c]], rows_v.at[b], sems[b]
            ).wait()
            pltpu.async_copy(rows_v.at[b], out_slice(c), sems[b])
        for b in range(_NBUF):
            c = (n_groups - 1) * _NBUF + b
            pltpu.make_async_copy(rows_v.at[b], out_slice(c), sems[b]).wait()

    return k(weight, idx3)


def kernel(x, weight):
    batch, num_fields = x.shape
    b_total = batch * num_fields
    b_per_w = b_total // _NW
    n_chunks = b_per_w // _CHUNK
    idx3 = x.reshape(_NW, n_chunks, _CHUNK).astype(jnp.int32)
    out = _sc_embedding_lookup(idx3, weight, n_chunks)
    return out.reshape(batch, num_fields, weight.shape[1])


# speedup vs baseline: 1.5771x; 1.0044x over previous
import functools

import jax
import jax.numpy as jnp
from jax import lax
from jax.experimental import pallas as pl
from jax.experimental.pallas import tpu as pltpu
from jax.experimental.pallas import tpu_sc as plsc

_D = 32
_CHUNK = 128
_NBUF = 8
_NW = 32


def _sc_embedding_lookup(idx3, weight, n_chunks):
    b_per_w = n_chunks * _CHUNK
    b_total = _NW * b_per_w
    n_groups = n_chunks // _NBUF

    mesh = plsc.VectorSubcoreMesh(core_axis_name="c", subcore_axis_name="s")

    @functools.partial(
        pl.kernel,
        mesh=mesh,
        out_type=jax.ShapeDtypeStruct((b_total, _D), jnp.float32),
        scratch_types=[
            pltpu.VMEM((n_chunks, _CHUNK), jnp.int32),
            pltpu.VMEM((_NBUF, _CHUNK, _D), jnp.float32),
        ] + [pltpu.SemaphoreType.DMA] * _NBUF,
        compiler_params=pltpu.CompilerParams(use_tc_tiling_on_sc=False),
    )
    def k(table_hbm, idx_hbm, out_hbm, idx_v, rows_v, *sems):
        wid = lax.axis_index("s") * 2 + lax.axis_index("c")
        base = wid * b_per_w
        pltpu.sync_copy(idx_hbm.at[wid], idx_v)

        def out_slice(c):
            return out_hbm.at[pl.ds(base + c * _CHUNK, _CHUNK)]

        for b in range(_NBUF):
            pltpu.async_copy(table_hbm.at[idx_v.at[b]], rows_v.at[b], sems[b])

        def body(g, carry):
            for b in range(_NBUF):
                c = g * _NBUF + b
                pltpu.make_async_copy(
                    table_hbm.at[idx_v.at[c]], rows_v.at[b], sems[b]
                ).wait()
                pltpu.async_copy(rows_v.at[b], out_slice(c), sems[b])
                pltpu.make_async_copy(rows_v.at[b], out_slice(c), sems[b]).wait()
                pltpu.async_copy(
                    table_hbm.at[idx_v.at[c + _NBUF]], rows_v.at[b], sems[b]
                )
            return carry

        lax.fori_loop(0, n_groups - 1, body, 0)

        for b in range(_NBUF):
            c = (n_groups - 1) * _NBUF + b
            pltpu.make_async_copy(
                table_hbm.at[idx_v.at[c]], rows_v.at[b], sems[b]
            ).wait()
            pltpu.async_copy(rows_v.at[b], out_slice(c), sems[b])
        for b in range(_NBUF):
            c = (n_groups - 1) * _NBUF + b
            pltpu.make_async_copy(rows_v.at[b], out_slice(c), sems[b]).wait()

    return k(weight, idx3)


def kernel(x, weight):
    batch, num_fields = x.shape
    b_total = batch * num_fields
    b_per_w = b_total // _NW
    n_chunks = b_per_w // _CHUNK
    idx3 = x.reshape(_NW, n_chunks, _CHUNK).astype(jnp.int32)
    out = _sc_embedding_lookup(idx3, weight, n_chunks)
    return out.reshape(batch, num_fields, weight.shape[1])


# speedup vs baseline: 2.2182x; 1.4065x over previous
import functools

import jax
import jax.numpy as jnp
from jax import lax
from jax.experimental import pallas as pl
from jax.experimental.pallas import tpu as pltpu
from jax.experimental.pallas import tpu_sc as plsc

_V = 1000000
_D = 32
_NW = 32
_L = 16
_CHUNK = 128
_NBUF = 8
_LB = 8192
_OB = _LB * _D // 128
_GRID = (_V + _LB - 1) // _LB


def _tc_transpose(wt):
    strip = _LB // 4

    def body(x_ref, o_ref):
        x = x_ref[...]
        eye = jnp.eye(_D, dtype=jnp.float32)
        for q in range(4):
            piece = lax.dot_general(
                x[:, strip * q:strip * (q + 1)], eye,
                (((0,), (0,)), ((), ())),
                preferred_element_type=jnp.float32,
            )
            o_ref[:, _D * q:_D * (q + 1)] = piece

    return pl.pallas_call(
        body,
        grid=(_GRID,),
        in_specs=[pl.BlockSpec((_D, _LB), lambda i: (0, i))],
        out_specs=pl.BlockSpec((_OB, 128), lambda i: (i, 0)),
        out_shape=jax.ShapeDtypeStruct((_GRID * _OB, 128), jnp.float32),
    )(wt)


def _sc_gather(idx3, table, n_chunks):
    b_per_w = n_chunks * _CHUNK
    b_total = _NW * b_per_w
    n_groups = n_chunks // _NBUF

    mesh = plsc.VectorSubcoreMesh(core_axis_name="c", subcore_axis_name="s")

    @functools.partial(
        pl.kernel,
        mesh=mesh,
        out_type=jax.ShapeDtypeStruct((b_total, _D), jnp.float32),
        scratch_types=[
            pltpu.VMEM((n_chunks, _CHUNK), jnp.int32),
            pltpu.VMEM((_NBUF, _CHUNK, _D), jnp.float32),
        ] + [pltpu.SemaphoreType.DMA] * _NBUF,
        compiler_params=pltpu.CompilerParams(use_tc_tiling_on_sc=False),
    )
    def k(table_hbm, idx_hbm, out_hbm, idx_v, rows_v, *sems):
        wid = lax.axis_index("s") * 2 + lax.axis_index("c")
        base = wid * b_per_w
        pltpu.sync_copy(idx_hbm.at[wid], idx_v)

        sh = _LB.bit_length() - 3
        def remap_body(c, carry):
            for o in range(0, _CHUNK, _L):
                v = idx_v[c, pl.ds(o, _L)]
                k_ = ((v & ~(_LB - 1))
                      | ((v & (_LB // 4 - 1)) << 2)
                      | ((v & (_LB - 1)) >> sh))
                idx_v[c, pl.ds(o, _L)] = k_
            return carry

        lax.fori_loop(0, n_chunks, remap_body, 0)

        def out_slice(c):
            return out_hbm.at[pl.ds(base + c * _CHUNK, _CHUNK)]

        for b in range(_NBUF):
            pltpu.async_copy(table_hbm.at[idx_v.at[b]], rows_v.at[b], sems[b])

        def body(g, carry):
            for b in range(_NBUF):
                c = g * _NBUF + b
                pltpu.make_async_copy(
                    table_hbm.at[idx_v.at[c]], rows_v.at[b], sems[b]
                ).wait()
                pltpu.async_copy(rows_v.at[b], out_slice(c), sems[b])
                pltpu.make_async_copy(rows_v.at[b], out_slice(c), sems[b]).wait()
                pltpu.async_copy(
                    table_hbm.at[idx_v.at[c + _NBUF]], rows_v.at[b], sems[b]
                )
            return carry

        lax.fori_loop(0, n_groups - 1, body, 0)

        for b in range(_NBUF):
            c = (n_groups - 1) * _NBUF + b
            pltpu.make_async_copy(
                table_hbm.at[idx_v.at[c]], rows_v.at[b], sems[b]
            ).wait()
            pltpu.async_copy(rows_v.at[b], out_slice(c), sems[b])
        for b in range(_NBUF):
            c = (n_groups - 1) * _NBUF + b
            pltpu.make_async_copy(rows_v.at[b], out_slice(c), sems[b]).wait()

    return k(table, idx3)


def kernel(x, weight):
    batch, num_fields = x.shape
    b_total = batch * num_fields
    n_chunks = b_total // _NW // _CHUNK
    w2 = _tc_transpose(weight.T)
    table = w2.reshape(_GRID * _LB, _D)
    idx3 = x.reshape(_NW, n_chunks, _CHUNK).astype(jnp.int32)
    out = _sc_gather(idx3, table, n_chunks)
    return out.reshape(batch, num_fields, weight.shape[1])


# speedup vs baseline: 2.2443x; 1.0118x over previous
import functools

import jax
import jax.numpy as jnp
from jax import lax
from jax.experimental import pallas as pl
from jax.experimental.pallas import tpu as pltpu
from jax.experimental.pallas import tpu_sc as plsc

_V = 1000000
_D = 32
_NW = 32
_L = 16
_CHUNK = 128
_NBUF = 8
_LB = 32768
_OB = _LB * _D // 128
_GRID = (_V + _LB - 1) // _LB


def _tc_transpose(wt):
    strip = _LB // 4

    def body(x_ref, o_ref):
        x = x_ref[...]
        eye = jnp.eye(_D, dtype=jnp.float32)
        for q in range(4):
            piece = lax.dot_general(
                x[:, strip * q:strip * (q + 1)], eye,
                (((0,), (0,)), ((), ())),
                preferred_element_type=jnp.float32,
            )
            o_ref[:, _D * q:_D * (q + 1)] = piece

    return pl.pallas_call(
        body,
        grid=(_GRID,),
        in_specs=[pl.BlockSpec((_D, _LB), lambda i: (0, i))],
        out_specs=pl.BlockSpec((_OB, 128), lambda i: (i, 0)),
        out_shape=jax.ShapeDtypeStruct((_GRID * _OB, 128), jnp.float32),
    )(wt)


def _sc_gather(idx3, table, n_chunks):
    b_per_w = n_chunks * _CHUNK
    b_total = _NW * b_per_w
    n_groups = n_chunks // _NBUF

    mesh = plsc.VectorSubcoreMesh(core_axis_name="c", subcore_axis_name="s")

    @functools.partial(
        pl.kernel,
        mesh=mesh,
        out_type=jax.ShapeDtypeStruct((b_total, _D), jnp.float32),
        scratch_types=[
            pltpu.VMEM((n_chunks, _CHUNK), jnp.int32),
            pltpu.VMEM((_NBUF, _CHUNK, _D), jnp.float32),
        ] + [pltpu.SemaphoreType.DMA] * _NBUF,
        compiler_params=pltpu.CompilerParams(use_tc_tiling_on_sc=False),
    )
    def k(table_hbm, idx_hbm, out_hbm, idx_v, rows_v, *sems):
        wid = lax.axis_index("s") * 2 + lax.axis_index("c")
        base = wid * b_per_w
        pltpu.sync_copy(idx_hbm.at[wid], idx_v)

        sh = _LB.bit_length() - 3
        def remap_body(c, carry):
            for o in range(0, _CHUNK, _L):
                v = idx_v[c, pl.ds(o, _L)]
                k_ = ((v & ~(_LB - 1))
                      | ((v & (_LB // 4 - 1)) << 2)
                      | ((v & (_LB - 1)) >> sh))
                idx_v[c, pl.ds(o, _L)] = k_
            return carry

        lax.fori_loop(0, n_chunks, remap_body, 0)

        def out_slice(c):
            return out_hbm.at[pl.ds(base + c * _CHUNK, _CHUNK)]

        for b in range(_NBUF):
            pltpu.async_copy(table_hbm.at[idx_v.at[b]], rows_v.at[b], sems[b])

        def body(g, carry):
            for b in range(_NBUF):
                c = g * _NBUF + b
                pltpu.make_async_copy(
                    table_hbm.at[idx_v.at[c]], rows_v.at[b], sems[b]
                ).wait()
                pltpu.async_copy(rows_v.at[b], out_slice(c), sems[b])
                pltpu.make_async_copy(rows_v.at[b], out_slice(c), sems[b]).wait()
                pltpu.async_copy(
                    table_hbm.at[idx_v.at[c + _NBUF]], rows_v.at[b], sems[b]
                )
            return carry

        lax.fori_loop(0, n_groups - 1, body, 0)

        for b in range(_NBUF):
            c = (n_groups - 1) * _NBUF + b
            pltpu.make_async_copy(
                table_hbm.at[idx_v.at[c]], rows_v.at[b], sems[b]
            ).wait()
            pltpu.async_copy(rows_v.at[b], out_slice(c), sems[b])
        for b in range(_NBUF):
            c = (n_groups - 1) * _NBUF + b
            pltpu.make_async_copy(rows_v.at[b], out_slice(c), sems[b]).wait()

    return k(table, idx3)


def kernel(x, weight):
    batch, num_fields = x.shape
    b_total = batch * num_fields
    n_chunks = b_total // _NW // _CHUNK
    w2 = _tc_transpose(weight.T)
    table = w2.reshape(_GRID * _LB, _D)
    idx3 = x.reshape(_NW, n_chunks, _CHUNK).astype(jnp.int32)
    out = _sc_gather(idx3, table, n_chunks)
    return out.reshape(batch, num_fields, weight.shape[1])
